# baseline (speedup 1.0000x reference)
import jax
import jax.numpy as jnp
from jax.experimental import pallas as pl
from jax.experimental.pallas import tpu as pltpu

_ANCHOR_W = (116.0, 156.0, 373.0)
_ANCHOR_H = (90.0, 198.0, 326.0)
_A = 3
_CLS = 80
_CHUNKS = tuple(
    (a, off, 32) for a in range(3) for off in (0, 32, 64, 96)
)


def _compute(buf, ob, a, off, sz):
    n = buf.shape[0]
    w = buf.shape[3]
    gx = jax.lax.broadcasted_iota(jnp.int32, (sz, w), 1).astype(jnp.float32)
    gy = jax.lax.broadcasted_iota(jnp.int32, (sz, w), 0).astype(jnp.float32) + float(off)
    bx = jax.nn.sigmoid(buf[:, 0, 0:sz]) + gx[None]
    by = jax.nn.sigmoid(buf[:, 1, 0:sz]) + gy[None]
    bw = buf[:, 2, 0:sz] * _ANCHOR_W[a]
    bh = buf[:, 3, 0:sz] * _ANCHOR_H[a]
    obj = buf[:, 4, 0:sz]
    logits = buf[:, 5 : 5 + _CLS, 0:sz]
    score = logits * obj[:, None]
    s = score.reshape(n * _CLS, sz, w)
    smax = jnp.max(s, axis=0)
    idx = jax.lax.broadcasted_iota(jnp.int32, (n * _CLS, sz, w), 0).astype(
        jnp.float32
    )
    sarg = jnp.min(
        jnp.where(s == smax[None], idx, jnp.float32(n * _CLS)), axis=0
    )
    ob[:, 0, 0:sz] = bx
    ob[:, 1, 0:sz] = by
    ob[:, 2, 0:sz] = bw
    ob[:, 3, 0:sz] = bh
    ob[:, 4, 0:sz] = jnp.broadcast_to(smax[None], (n, sz, w))
    ob[:, 5, 0:sz] = jnp.broadcast_to(sarg[None], (n, sz, w))


def _decode_kernel(x_hbm, o_hbm, b0, b1, ob0, ob1, insem, outsem):
    bufs = (b0, b1)
    obufs = (ob0, ob1)

    ca = 5 + _CLS

    def in_copy(k):
        a, off, sz = _CHUNKS[k]
        return pltpu.make_async_copy(
            x_hbm.at[:, pl.ds(a * ca, ca), pl.ds(off, sz), :],
            bufs[k % 2].at[:, :, pl.ds(0, sz), :],
            insem.at[k % 2],
        )

    def out_copy(k):
        a, off, sz = _CHUNKS[k]
        return pltpu.make_async_copy(
            obufs[k % 2].at[:, :, pl.ds(0, sz), :],
            o_hbm.at[:, pl.ds(a * 6, 6), pl.ds(off, sz), :],
            outsem.at[k % 2],
        )

    in_copy(0).start()
    in_copy(1).start()
    nk = len(_CHUNKS)
    for k in range(nk):
        a, off, sz = _CHUNKS[k]
        slot = k % 2
        in_copy(k).wait()
        if k >= 2:
            out_copy(k - 2).wait()
        _compute(bufs[slot], obufs[slot], a, off, sz)
        out_copy(k).start()
        if k + 2 < nk:
            in_copy(k + 2).start()
    out_copy(nk - 2).wait()
    out_copy(nk - 1).wait()


def kernel(x):
    n, c, h, w = x.shape
    return pl.pallas_call(
        _decode_kernel,
        in_specs=[pl.BlockSpec(memory_space=pl.ANY)],
        out_specs=pl.BlockSpec(memory_space=pl.ANY),
        out_shape=jax.ShapeDtypeStruct((n, _A * 6, h, w), x.dtype),
        scratch_shapes=[
            pltpu.VMEM((n, 5 + _CLS, 32, w), jnp.float32),
            pltpu.VMEM((n, 5 + _CLS, 32, w), jnp.float32),
            pltpu.VMEM((n, 6, 32, w), jnp.float32),
            pltpu.VMEM((n, 6, 32, w), jnp.float32),
            pltpu.SemaphoreType.DMA((2,)),
            pltpu.SemaphoreType.DMA((2,)),
        ],
        compiler_params=pltpu.CompilerParams(
            vmem_limit_bytes=64 * 1024 * 1024,
        ),
        name="anchor_decode_manual",
    )(x)


# speedup vs baseline: 1.0507x; 1.0507x over previous
import jax
import jax.numpy as jnp
from jax.experimental import pallas as pl
from jax.experimental.pallas import tpu as pltpu

_ANCHOR_W = (116.0, 156.0, 373.0)
_ANCHOR_H = (90.0, 198.0, 326.0)
_A = 3
_CLS = 80
_HB = 16


def _decode_kernel(x_ref, o_ref):
    n, _, hb, w = x_ref.shape
    h0 = (pl.program_id(0) * hb).astype(jnp.float32)
    gx = jax.lax.broadcasted_iota(jnp.int32, (hb, w), 1).astype(jnp.float32)
    gy = jax.lax.broadcasted_iota(jnp.int32, (hb, w), 0).astype(jnp.float32) + h0
    for a in range(_A):
        base = a * (5 + _CLS)
        bx = jax.nn.sigmoid(x_ref[:, base + 0]) + gx[None]
        by = jax.nn.sigmoid(x_ref[:, base + 1]) + gy[None]
        bw = x_ref[:, base + 2] * _ANCHOR_W[a]
        bh = x_ref[:, base + 3] * _ANCHOR_H[a]
        obj = x_ref[:, base + 4]
        logits = x_ref[:, base + 5 : base + 5 + _CLS]
        score = logits * obj[:, None]
        s = score.reshape(n * _CLS, hb, w)
        smax = jnp.max(s, axis=0)
        idx = jax.lax.broadcasted_iota(jnp.int32, (n * _CLS, hb, w), 0).astype(
            jnp.float32
        )
        sarg = jnp.min(
            jnp.where(s == smax[None], idx, jnp.float32(n * _CLS)), axis=0
        )
        o_ref[:, a * 6 + 0] = bx
        o_ref[:, a * 6 + 1] = by
        o_ref[:, a * 6 + 2] = bw
        o_ref[:, a * 6 + 3] = bh
        o_ref[:, a * 6 + 4] = jnp.broadcast_to(smax[None], (n, hb, w))
        o_ref[:, a * 6 + 5] = jnp.broadcast_to(sarg[None], (n, hb, w))


def kernel(x):
    n, c, h, w = x.shape
    return pl.pallas_call(
        _decode_kernel,
        grid=(h // _HB,),
        in_specs=[pl.BlockSpec((n, c, _HB, w), lambda i: (0, 0, i, 0))],
        out_specs=pl.BlockSpec((n, _A * 6, _HB, w), lambda i: (0, 0, i, 0)),
        out_shape=jax.ShapeDtypeStruct((n, _A * 6, h, w), x.dtype),
        compiler_params=pltpu.CompilerParams(
            dimension_semantics=("parallel",),
            vmem_limit_bytes=64 * 1024 * 1024,
        ),
        name="anchor_decode",
    )(x)
